# baseline (speedup 1.0000x reference)
import functools

import jax
import jax.numpy as jnp
from jax import lax
from jax.experimental import pallas as pl
from jax.experimental.pallas import tpu as pltpu
from jax.experimental.pallas import tpu_sc as plsc

NC, NS = 2, 16
D = 64
V = 200
CH = 400
ROUND = NS * CH
U = 16


@functools.partial(jax.jit, static_argnums=(2,))
def _lookup(pos_flat, tab_flat, B):
    half = B // NC
    n_rounds = half // ROUND

    mesh = plsc.VectorSubcoreMesh(
        core_axis_name="c", subcore_axis_name="s",
        num_cores=NC, num_subcores=NS)

    @functools.partial(
        pl.kernel,
        out_type=jax.ShapeDtypeStruct((B * D,), jnp.float32),
        mesh=mesh,
        scratch_types=[
            pltpu.VMEM((V * D,), jnp.float32),
            pltpu.VMEM((2, CH), jnp.int32),
            pltpu.VMEM((CH * D,), jnp.float32),
            pltpu.VMEM_SHARED((2, ROUND * D), jnp.float32),
            pltpu.SemaphoreType.DMA((2,)),
            pltpu.SemaphoreType.DMA((2,)),
        ],
        compiler_params=pltpu.CompilerParams(
            use_tc_tiling_on_sc=False, needs_layout_passes=False),
    )
    def k(pos_hbm, tab_hbm, out_hbm, tab_v, idx_v, slot, shared, isem, osem):
        sid = lax.axis_index("s")
        cid = lax.axis_index("c")
        base = cid * half

        pltpu.sync_copy(tab_hbm, tab_v)
        iota = lax.iota(jnp.int32, U)
        iota_d = iota * D

        def idx_off(r):
            return base + r * ROUND + sid * CH

        def fire_idx(r, h):
            pltpu.async_copy(
                pos_hbm.at[pl.ds(idx_off(r), CH)], idx_v.at[h], isem.at[h])

        def wait_idx(r, h):
            pltpu.make_async_copy(
                pos_hbm.at[pl.ds(idx_off(r), CH)], idx_v.at[h],
                isem.at[h]).wait()

        def fire_out(r, h):
            pltpu.async_copy(
                shared.at[h],
                out_hbm.at[pl.ds((base + r * ROUND) * D, ROUND * D)],
                osem.at[h])

        def wait_out(r, h):
            pltpu.make_async_copy(
                shared.at[h],
                out_hbm.at[pl.ds((base + r * ROUND) * D, ROUND * D)],
                osem.at[h]).wait()

        fire_idx(0, 0)

        def round_body(r, carry):
            h = lax.rem(r, 2)
            wait_idx(r, h)

            @pl.when(r < n_rounds - 1)
            def _():
                fire_idx(r + 1, 1 - h)

            idx_h = idx_v.at[h]

            def row_body(rr, carry):
                rbase = rr * U
                iv = idx_h[pl.ds(rbase, U)] * D
                ov = rbase * D + iota_d

                def col_body(q, carry):
                    q16 = q * 16
                    ivq = iv + q16
                    ovq = ov + q16
                    for c in range(16):
                        dc = (iota + c) & 15
                        plsc.store_scatter(
                            slot, [ovq + dc],
                            plsc.load_gather(tab_v, [ivq + dc]))
                    return carry

                lax.fori_loop(0, D // 16, col_body, carry)
                return carry

            lax.fori_loop(0, CH // U, row_body, carry)

            @pl.when(jnp.logical_and(sid == 0, r >= 2))
            def _():
                wait_out(r - 2, h)

            plsc.subcore_barrier()
            pltpu.sync_copy(slot, shared.at[h, pl.ds(sid * CH * D, CH * D)])
            plsc.subcore_barrier()

            @pl.when(sid == 0)
            def _():
                fire_out(r, h)

            return carry

        lax.fori_loop(0, n_rounds, round_body, 0)

        @pl.when(sid == 0)
        def _():
            wait_out(n_rounds - 2, lax.rem(n_rounds - 2, 2))
            wait_out(n_rounds - 1, lax.rem(n_rounds - 1, 2))

    return k(pos_flat, tab_flat)


def kernel(positions, table):
    batch, seq = positions.shape
    b = batch * seq
    pos_flat = positions.reshape(b).astype(jnp.int32)
    out = _lookup(pos_flat, table.reshape(V * D), b)
    return out.reshape(batch, seq, D)


# speedup vs baseline: 1.0711x; 1.0711x over previous
import functools

import jax
import jax.numpy as jnp
from jax import lax
from jax.experimental import pallas as pl
from jax.experimental.pallas import tpu as pltpu
from jax.experimental.pallas import tpu_sc as plsc

NC, NS = 2, 16
NW = NC * NS
D = 64
V = 200
CH = 400
SB = 12800
NSLOT = 4
U = 16


@functools.partial(jax.jit, static_argnums=(2,))
def _lookup(pos_flat, tab_flat, B):
    per_w = B // NW
    n_sb = per_w // SB
    n_ch = SB // CH

    mesh = plsc.VectorSubcoreMesh(
        core_axis_name="c", subcore_axis_name="s",
        num_cores=NC, num_subcores=NS)

    @functools.partial(
        pl.kernel,
        out_type=jax.ShapeDtypeStruct((B * D,), jnp.float32),
        mesh=mesh,
        scratch_types=[
            pltpu.VMEM((V * D,), jnp.float32),
            pltpu.VMEM((SB,), jnp.int32),
            pltpu.VMEM((NSLOT, CH * D), jnp.float32),
            pltpu.SemaphoreType.DMA((NSLOT,)),
        ],
        compiler_params=pltpu.CompilerParams(
            use_tc_tiling_on_sc=False, needs_layout_passes=False),
    )
    def k(pos_hbm, tab_hbm, out_hbm, tab_v, idx_v, rows_v, osem):
        wid = lax.axis_index("s") * NC + lax.axis_index("c")
        base = wid * per_w

        pltpu.sync_copy(tab_hbm, tab_v)
        iota = lax.iota(jnp.int32, U)
        iota_d = iota * D

        def wait_write(sb_base, g, s):
            pltpu.make_async_copy(
                rows_v.at[s],
                out_hbm.at[pl.ds((sb_base + g * CH) * D, CH * D)],
                osem.at[s]).wait()

        def sb_body(sbi, carry):
            sb_base = base + sbi * SB
            pltpu.sync_copy(pos_hbm.at[pl.ds(sb_base, SB)], idx_v)

            def g_body(g, carry):
                s = lax.rem(g, NSLOT)
                goff = g * CH

                @pl.when(g >= NSLOT)
                def _():
                    wait_write(sb_base, g - NSLOT, s)

                slot = rows_v.at[s]

                def row_body(r, carry):
                    rbase = r * U
                    iv = idx_v[pl.ds(goff + rbase, U)] * D
                    ov = rbase * D + iota_d

                    def col_body(q, carry):
                        q16 = q * 16
                        ivq = iv + q16
                        ovq = ov + q16
                        for c in range(16):
                            dc = (iota + c) & 15
                            plsc.store_scatter(
                                slot, [ovq + dc],
                                plsc.load_gather(tab_v, [ivq + dc]))
                        return carry

                    lax.fori_loop(0, D // 16, col_body, carry)
                    return carry

                lax.fori_loop(0, CH // U, row_body, carry)

                pltpu.async_copy(
                    slot,
                    out_hbm.at[pl.ds((sb_base + goff) * D, CH * D)],
                    osem.at[s])
                return carry

            lax.fori_loop(0, n_ch, g_body, carry)

            for g in range(n_ch - NSLOT, n_ch):
                wait_write(sb_base, g, lax.rem(g, NSLOT))
            return carry

        lax.fori_loop(0, n_sb, sb_body, 0)

    return k(pos_flat, tab_flat)


def kernel(positions, table):
    batch, seq = positions.shape
    b = batch * seq
    pos_flat = positions.reshape(b).astype(jnp.int32)
    out = _lookup(pos_flat, table.reshape(V * D), b)
    return out.reshape(batch, seq, D)


# speedup vs baseline: 1.0995x; 1.0265x over previous
import functools

import jax
import jax.numpy as jnp
from jax import lax
from jax.experimental import pallas as pl
from jax.experimental.pallas import tpu as pltpu
from jax.experimental.pallas import tpu_sc as plsc

NC, NS = 2, 16
NW = NC * NS
D = 64
V = 200
CH = 512
SB = 12800
NSLOT = 3
U = 16


@functools.partial(jax.jit, static_argnums=(2,))
def _lookup(pos_flat, tab_flat, B):
    per_w = B // NW
    n_sb = per_w // SB
    n_ch = SB // CH

    mesh = plsc.VectorSubcoreMesh(
        core_axis_name="c", subcore_axis_name="s",
        num_cores=NC, num_subcores=NS)

    @functools.partial(
        pl.kernel,
        out_type=jax.ShapeDtypeStruct((B * D,), jnp.float32),
        mesh=mesh,
        scratch_types=[
            pltpu.VMEM((V * D,), jnp.float32),
            pltpu.VMEM((SB,), jnp.int32),
            pltpu.VMEM((NSLOT, CH * D), jnp.float32),
            pltpu.SemaphoreType.DMA((NSLOT,)),
        ],
        compiler_params=pltpu.CompilerParams(
            use_tc_tiling_on_sc=False, needs_layout_passes=False),
    )
    def k(pos_hbm, tab_hbm, out_hbm, tab_v, idx_v, rows_v, osem):
        wid = lax.axis_index("s") * NC + lax.axis_index("c")
        base = wid * per_w

        pltpu.sync_copy(tab_hbm, tab_v)

        def wait_write(sb_base, g, s):
            pltpu.make_async_copy(
                rows_v.at[s],
                out_hbm.at[pl.ds((sb_base + g * CH) * D, CH * D)],
                osem.at[s]).wait()

        def sb_body(sbi, carry):
            sb_base = base + sbi * SB
            pltpu.sync_copy(pos_hbm.at[pl.ds(sb_base, SB)], idx_v)

            def g_body(g, carry):
                s = lax.rem(g, NSLOT)

                @pl.when(g >= NSLOT)
                def _():
                    wait_write(sb_base, g - NSLOT, s)

                slot = rows_v.at[s]
                goff = g * CH

                def row_body(r, carry):
                    rbase = r * U
                    iv = idx_v[pl.ds(goff + rbase, U)] * D
                    for u in range(U):
                        tb = iv[u]
                        ob = (rbase + u) * D
                        for kk in range(D // 16):
                            slot[pl.ds(ob + kk * 16, 16)] = (
                                tab_v[pl.ds(tb + kk * 16, 16)])
                    return carry

                lax.fori_loop(0, CH // U, row_body, carry)

                pltpu.async_copy(
                    slot,
                    out_hbm.at[pl.ds((sb_base + goff) * D, CH * D)],
                    osem.at[s])
                return carry

            lax.fori_loop(0, n_ch, g_body, carry)

            for g in range(n_ch - NSLOT, n_ch):
                wait_write(sb_base, g, lax.rem(g, NSLOT))
            return carry

        lax.fori_loop(0, n_sb, sb_body, 0)

    return k(pos_flat, tab_flat)


def kernel(positions, table):
    batch, seq = positions.shape
    b = batch * seq
    pos_flat = positions.reshape(b).astype(jnp.int32)
    out = _lookup(pos_flat, table.reshape(V * D), b)
    return out.reshape(batch, seq, D)
